# baseline (speedup 1.0000x reference)
import functools

import jax
import jax.numpy as jnp
from jax import lax
from jax.experimental import pallas as pl
from jax.experimental.pallas import tpu as pltpu
from jax.experimental.pallas import tpu_sc as plsc

BATCH = 4
SEQ = 8192
D = 1024

SEQ_TC = 7168
SEQ_SC = SEQ - SEQ_TC
SEQ_BLK = 1024

NC = 2
NS = 16
NW = NC * NS

ROWS_PER_W = SEQ_SC // NW
R = 16
CHUNKS = ROWS_PER_W // R
CHUNK_ELEMS = R * D
VECS = CHUNK_ELEMS // 16
STEPS = CHUNKS * BATCH

_mesh = plsc.VectorSubcoreMesh(core_axis_name="c", subcore_axis_name="s")


@functools.partial(
    pl.kernel,
    mesh=_mesh,
    out_type=jax.ShapeDtypeStruct((BATCH * SEQ_SC * D,), jnp.float32),
    scratch_types=[
        pltpu.VMEM((CHUNK_ELEMS,), jnp.float32),
        pltpu.VMEM((CHUNK_ELEMS,), jnp.float32),
        pltpu.VMEM((CHUNK_ELEMS,), jnp.float32),
        pltpu.VMEM((CHUNK_ELEMS,), jnp.float32),
        pltpu.VMEM((CHUNK_ELEMS,), jnp.float32),
        pltpu.VMEM((CHUNK_ELEMS,), jnp.float32),
        pltpu.SemaphoreType.DMA,
        pltpu.SemaphoreType.DMA,
        pltpu.SemaphoreType.DMA,
        pltpu.SemaphoreType.DMA,
        pltpu.SemaphoreType.DMA,
        pltpu.SemaphoreType.DMA,
        pltpu.SemaphoreType.DMA,
        pltpu.SemaphoreType.DMA,
        pltpu.SemaphoreType.DMA,
        pltpu.SemaphoreType.DMA,
    ],
)
def _sc_add(x_hbm, pos_hbm, out_hbm,
            xb0, xb1, xb2, xb3, pb0, pb1,
            in0, in1, in2, in3, o0, o1, o2, o3, ps0, ps1):
    wid = lax.axis_index("s") * NC + lax.axis_index("c")
    row0 = SEQ_TC + wid * ROWS_PER_W
    out_row0 = wid * ROWS_PER_W

    xbufs = (xb0, xb1, xb2, xb3)
    pbufs = (pb0, pb1)
    in_sems = (in0, in1, in2, in3)
    out_sems = (o0, o1, o2, o3)
    pos_sems = (ps0, ps1)

    def x_off(step):
        c, b = step // BATCH, step % BATCH
        return (b * SEQ + row0 + c * R) * D

    def out_off(step):
        c, b = step // BATCH, step % BATCH
        return (b * SEQ_SC + out_row0 + c * R) * D

    def start_x(step):
        p = step % 4
        return pltpu.async_copy(
            x_hbm.at[pl.ds(x_off(step), CHUNK_ELEMS)], xbufs[p], in_sems[p])

    def start_pos(c):
        p = c % 2
        return pltpu.async_copy(
            pos_hbm.at[pl.ds((row0 + c * R) * D, CHUNK_ELEMS)],
            pbufs[p], pos_sems[p])

    def start_out(step):
        p = step % 4
        return pltpu.async_copy(
            xbufs[p], out_hbm.at[pl.ds(out_off(step), CHUNK_ELEMS)],
            out_sems[p])

    in_flight = {}
    pos_flight = {}
    out_flight = {}

    pos_flight[0] = start_pos(0)
    in_flight[0] = start_x(0)
    in_flight[1] = start_x(1)

    for s in range(STEPS):
        p = s % 4
        c = s // BATCH
        nxt = s + 2
        if nxt < STEPS:
            if nxt - 4 >= 0:
                out_flight[nxt - 4].wait()
            in_flight[nxt] = start_x(nxt)
        if s % BATCH == 3 and c + 1 < CHUNKS:
            pos_flight[c + 1] = start_pos(c + 1)

        in_flight[s].wait()
        if s % BATCH == 0:
            pos_flight[c].wait()

        xb = xbufs[p]
        pb = pbufs[c % 2]

        @plsc.parallel_loop(0, VECS, step=1, unroll=8)
        def _add(k, xb=xb, pb=pb):
            sl = pl.ds(k * 16, 16)
            xb[sl] = xb[sl] + pb[sl]

        out_flight[s] = start_out(s)

    for s in range(max(0, STEPS - 4), STEPS):
        out_flight[s].wait()


def _tc_body(x_ref, pos_ref, o_ref):
    o_ref[...] = x_ref[...] + pos_ref[...]


def _tc_add(x_tc, pos_tc):
    n_s = SEQ_TC // SEQ_BLK
    return pl.pallas_call(
        _tc_body,
        grid=(n_s, BATCH),
        in_specs=[
            pl.BlockSpec((1, SEQ_BLK, D), lambda s, b: (b, s, 0)),
            pl.BlockSpec((SEQ_BLK, D), lambda s, b: (s, 0)),
        ],
        out_specs=pl.BlockSpec((1, SEQ_BLK, D), lambda s, b: (b, s, 0)),
        out_shape=jax.ShapeDtypeStruct((BATCH, SEQ_TC, D), jnp.float32),
    )(x_tc, pos_tc)


def kernel(x, pos_table):
    sc_out = _sc_add(x.reshape(-1), pos_table.reshape(-1))
    tc_out = _tc_add(x[:, :SEQ_TC, :], pos_table[:SEQ_TC, :])
    return jnp.concatenate(
        [tc_out, sc_out.reshape(BATCH, SEQ_SC, D)], axis=1)


# speedup vs baseline: 4.6678x; 4.6678x over previous
import jax
import jax.numpy as jnp
from jax.experimental import pallas as pl
from jax.experimental.pallas import tpu as pltpu

SEQ_BLK = 2048


def _add_kernel(x_ref, pos_ref, o_ref):
    o_ref[...] = x_ref[...] + pos_ref[...]


def kernel(x, pos_table):
    batch, seq, d = x.shape
    n_s = seq // SEQ_BLK
    return pl.pallas_call(
        _add_kernel,
        grid=(n_s, batch),
        in_specs=[
            pl.BlockSpec((1, SEQ_BLK, d), lambda s, b: (b, s, 0)),
            pl.BlockSpec((SEQ_BLK, d), lambda s, b: (s, 0)),
        ],
        out_specs=pl.BlockSpec((1, SEQ_BLK, d), lambda s, b: (b, s, 0)),
        out_shape=jax.ShapeDtypeStruct(x.shape, x.dtype),
        compiler_params=pltpu.CompilerParams(
            dimension_semantics=("parallel", "parallel")),
    )(x, pos_table)
